# baseline (speedup 1.0000x reference)
import jax
import jax.numpy as jnp
from jax import lax
from jax.experimental import pallas as pl
from jax.experimental.pallas import tpu as pltpu
from jax.experimental.pallas import tpu_sc as plsc

B = 16384
D = 64
F = 128
NC = 2
NS = 16
NW = NC * NS
BPW = B // NW
CH = 128
NCH = BPW // CH

NU = 1000000
NM = 100000
UROWS = 1 << 20
MROWS = 1 << 17
RCH = 512
U_PER_W = UROWS // NW
M_PER_W = MROWS // NW

_mesh = plsc.VectorSubcoreMesh(core_axis_name="c", subcore_axis_name="s")


def _repack_body(utab_hbm, mtab_hbm, uc_hbm, mc_hbm, pad_v, cmp_v, sem):
    wid = lax.axis_index("s") * NC + lax.axis_index("c")

    def do_chunk(tab_hbm, out_hbm, r0, rows):
        r0 = pl.multiple_of(r0, 8)
        pltpu.sync_copy(tab_hbm.at[pl.ds(r0, rows)], pad_v.at[pl.ds(0, rows)])

        def pack8(i, carry):
            for k in range(8):
                for j in range(D // 16):
                    cmp_v[i * 4 + k // 2, pl.ds((k % 2) * D + j * 16, 16)] = (
                        pad_v[i * 8 + k, pl.ds(j * 16, 16)])
            return carry

        lax.fori_loop(0, rows // 8, pack8, 0)
        k0 = pl.multiple_of(r0 // 2, 8)
        pltpu.sync_copy(cmp_v.at[pl.ds(0, rows // 2)],
                        out_hbm.at[pl.ds(k0, rows // 2)])

    ubase = wid * U_PER_W

    def uchunk(c, carry):
        r0 = ubase + c * RCH

        @pl.when(r0 + RCH <= NU)
        def _():
            do_chunk(utab_hbm, uc_hbm, r0, RCH)

        return carry

    lax.fori_loop(0, U_PER_W // RCH, uchunk, 0)

    @pl.when(ubase == (NU // U_PER_W) * U_PER_W)
    def _():
        t0 = (NU // RCH) * RCH
        do_chunk(utab_hbm, uc_hbm, t0, NU - t0)

    mbase = wid * M_PER_W

    def mchunk(c, carry):
        r0 = mbase + c * RCH

        @pl.when(r0 + RCH <= NM)
        def _():
            do_chunk(mtab_hbm, mc_hbm, r0, RCH)

        return carry

    lax.fori_loop(0, M_PER_W // RCH, mchunk, 0)

    @pl.when(mbase == (NM // M_PER_W) * M_PER_W)
    def _():
        t0 = (NM // RCH) * RCH
        do_chunk(mtab_hbm, mc_hbm, t0, NM - t0)


_repack = pl.kernel(
    _repack_body,
    mesh=_mesh,
    out_type=(
        jax.ShapeDtypeStruct((UROWS // 2, 2 * D), jnp.float32),
        jax.ShapeDtypeStruct((MROWS // 2, 2 * D), jnp.float32),
    ),
    scratch_types=[
        pltpu.VMEM((RCH, D), jnp.float32),
        pltpu.VMEM((RCH // 2, 2 * D), jnp.float32),
        pltpu.SemaphoreType.DMA,
    ],
)


def _dot_body(uidx_hbm, midx_hbm, fcw_hbm, uc_hbm, mc_hbm, out_hbm,
              uraw_v, mraw_v, udma_v, mdma_v, u_v, m_v, w_v, wb_v, o_v,
              sem_u, sem_m):
    wid = lax.axis_index("s") * NC + lax.axis_index("c")
    base = wid * BPW

    pltpu.sync_copy(fcw_hbm.at[0], w_v)
    for d in range(D):
        wv = w_v[pl.ds((d // 16) * 16, 16)]
        wb_v[d] = jnp.full((16,), wv[d % 16], jnp.float32)

    for c in range(NCH):
        pltpu.sync_copy(uidx_hbm.at[pl.ds(base + c * CH, CH)], uraw_v.at[c])
        pltpu.sync_copy(midx_hbm.at[pl.ds(base + c * CH, CH)], mraw_v.at[c])

    for c in range(NCH):
        for g in range(CH // 16):
            sl = pl.ds(g * 16, 16)
            udma_v[c, sl] = uraw_v[c, sl] >> 1
            mdma_v[c, sl] = mraw_v[c, sl] >> 1

    iota = lax.iota(jnp.int32, 16)

    for c in range(NCH):
        cu = pltpu.async_copy(uc_hbm.at[udma_v.at[c]], u_v, sem_u)
        cm = pltpu.async_copy(mc_hbm.at[mdma_v.at[c]], m_v, sem_m)
        cu.wait()
        cm.wait()

        def group(g, carry):
            i0 = jnp.full((16,), g * 16, jnp.int32) + iota
            uoff = (uraw_v[c, pl.ds(g * 16, 16)] & 1) << 6
            moff = (mraw_v[c, pl.ds(g * 16, 16)] & 1) << 6
            acc = jnp.zeros((16,), jnp.float32)
            for d in range(D):
                uu = plsc.load_gather(u_v, [i0, uoff + d])
                mm = plsc.load_gather(m_v, [i0, moff + d])
                acc = acc + uu * mm * wb_v[d]
            o_v[pl.ds(c * CH + g * 16, 16)] = acc
            return carry

        lax.fori_loop(0, CH // 16, group, 0)

    pltpu.sync_copy(o_v, out_hbm.at[pl.ds(base, BPW)])


_cf_dot = pl.kernel(
    _dot_body,
    mesh=_mesh,
    compiler_params=pltpu.CompilerParams(needs_layout_passes=False),
    out_type=jax.ShapeDtypeStruct((B,), jnp.float32),
    scratch_types=[
        pltpu.VMEM((NCH, CH), jnp.int32),
        pltpu.VMEM((NCH, CH), jnp.int32),
        pltpu.VMEM((NCH, CH), jnp.int32),
        pltpu.VMEM((NCH, CH), jnp.int32),
        pltpu.VMEM((CH, 2 * D), jnp.float32),
        pltpu.VMEM((CH, 2 * D), jnp.float32),
        pltpu.VMEM((192,), jnp.float32),
        pltpu.VMEM((D, 16), jnp.float32),
        pltpu.VMEM((BPW,), jnp.float32),
        pltpu.SemaphoreType.DMA,
        pltpu.SemaphoreType.DMA,
    ],
)

TB = 2048


def _tc_body(f_ref, w2_ref, b_ref, o_ref):
    o_ref[...] = jnp.sum(f_ref[...] * w2_ref[...], axis=1) + b_ref[0, 0]


_tc_feats = pl.pallas_call(
    _tc_body,
    grid=(B // TB,),
    in_specs=[
        pl.BlockSpec((TB, F), lambda i: (i, 0)),
        pl.BlockSpec((1, F), lambda i: (0, 0)),
        pl.BlockSpec((1, 1), lambda i: (0, 0)),
    ],
    out_specs=pl.BlockSpec((TB,), lambda i: (i,)),
    out_shape=jax.ShapeDtypeStruct((B,), jnp.float32),
)


def kernel(user_idx, movie_idx, movie_feats, user_table, movie_table, fc_w, fc_b):
    uc, mc = _repack(user_table, movie_table)
    cf = _cf_dot(user_idx, movie_idx, fc_w, uc, mc)
    w2 = fc_w[:, D:]
    b = fc_b.reshape(1, 1)
    content = _tc_feats(movie_feats, w2, b)
    return cf + content


# speedup vs baseline: 1.8118x; 1.8118x over previous
import jax
import jax.numpy as jnp
from jax import lax
from jax.experimental import pallas as pl
from jax.experimental.pallas import tpu as pltpu
from jax.experimental.pallas import tpu_sc as plsc

B = 16384
D = 64
F = 128
NC = 2
NS = 16
NW = NC * NS
BPW = B // NW
CH = 128
NCH = BPW // CH

_mesh = plsc.VectorSubcoreMesh(core_axis_name="c", subcore_axis_name="s")


def _dot_body(uidx_hbm, midx_hbm, fcw_hbm, utab_hbm, mtab_hbm, out_hbm,
              uidx_v, midx_v, u0_v, u1_v, m0_v, m1_v, w_v, wb_v, o_v,
              sem_u, sem_m):
    wid = lax.axis_index("s") * NC + lax.axis_index("c")
    base = wid * BPW

    pltpu.sync_copy(fcw_hbm.at[0], w_v)
    for d in range(D):
        wv = w_v[pl.ds((d // 16) * 16, 16)]
        wb_v[d] = jnp.full((16,), wv[d % 16], jnp.float32)

    for c in range(NCH):
        pltpu.sync_copy(uidx_hbm.at[pl.ds(base + c * CH, CH)], uidx_v.at[c])
        pltpu.sync_copy(midx_hbm.at[pl.ds(base + c * CH, CH)], midx_v.at[c])

    iota = lax.iota(jnp.int32, 16)

    bufs = [(u0_v, m0_v), (u1_v, m1_v)]

    def start(c, buf):
        ub, mb = bufs[buf]
        cu = pltpu.async_copy(utab_hbm.at[uidx_v.at[c]], ub, sem_u)
        cm = pltpu.async_copy(mtab_hbm.at[midx_v.at[c]], mb, sem_m)
        return cu, cm

    pend = start(0, 0)
    for c in range(NCH):
        if c + 1 < NCH:
            nxt = start(c + 1, (c + 1) % 2)
        pend[0].wait()
        pend[1].wait()
        ub, mb = bufs[c % 2]

        def group(g, carry):
            i0 = iota + g * 16
            acc = jnp.zeros((16,), jnp.float32)
            for d in range(D):
                i1 = jnp.full((16,), d, jnp.int32)
                uu = plsc.load_gather(ub, [i0, i1])
                mm = plsc.load_gather(mb, [i0, i1])
                acc = acc + uu * mm * wb_v[d]
            o_v[pl.ds(c * CH + g * 16, 16)] = acc
            return carry

        lax.fori_loop(0, CH // 16, group, 0)

        if c + 1 < NCH:
            pend = nxt

    pltpu.sync_copy(o_v, out_hbm.at[pl.ds(base, BPW)])


_cf_dot = pl.kernel(
    _dot_body,
    mesh=_mesh,
    compiler_params=pltpu.CompilerParams(
        use_tc_tiling_on_sc=False, needs_layout_passes=False),
    out_type=jax.ShapeDtypeStruct((B,), jnp.float32),
    scratch_types=[
        pltpu.VMEM((NCH, CH), jnp.int32),
        pltpu.VMEM((NCH, CH), jnp.int32),
        pltpu.VMEM((CH, D), jnp.float32),
        pltpu.VMEM((CH, D), jnp.float32),
        pltpu.VMEM((CH, D), jnp.float32),
        pltpu.VMEM((CH, D), jnp.float32),
        pltpu.VMEM((192,), jnp.float32),
        pltpu.VMEM((D, 16), jnp.float32),
        pltpu.VMEM((BPW,), jnp.float32),
        pltpu.SemaphoreType.DMA,
        pltpu.SemaphoreType.DMA,
    ],
)

TB = 2048


def _tc_body(f_ref, w2_ref, b_ref, o_ref):
    o_ref[...] = jnp.sum(f_ref[...] * w2_ref[...], axis=1) + b_ref[0, 0]


_tc_feats = pl.pallas_call(
    _tc_body,
    grid=(B // TB,),
    in_specs=[
        pl.BlockSpec((TB, F), lambda i: (i, 0)),
        pl.BlockSpec((1, F), lambda i: (0, 0)),
        pl.BlockSpec((1, 1), lambda i: (0, 0)),
    ],
    out_specs=pl.BlockSpec((TB,), lambda i: (i,)),
    out_shape=jax.ShapeDtypeStruct((B,), jnp.float32),
)


def kernel(user_idx, movie_idx, movie_feats, user_table, movie_table, fc_w, fc_b):
    cf = _cf_dot(user_idx, movie_idx, fc_w, user_table, movie_table)
    w2 = fc_w[:, D:]
    b = fc_b.reshape(1, 1)
    content = _tc_feats(movie_feats, w2, b)
    return cf + content


# speedup vs baseline: 1.8259x; 1.0078x over previous
import jax
import jax.numpy as jnp
from jax import lax
from jax.experimental import pallas as pl
from jax.experimental.pallas import tpu as pltpu
from jax.experimental.pallas import tpu_sc as plsc

B = 16384
D = 64
F = 128
NC = 2
NS = 16
NW = NC * NS
BPW = B // NW
CH = 128
NCH = BPW // CH
NU = 1000000
NM = 100000

_mesh = plsc.VectorSubcoreMesh(core_axis_name="c", subcore_axis_name="s")


def _dot_body(uidx_hbm, midx_hbm, fcw_hbm, uc_hbm, mc_hbm, out_hbm,
              uraw_v, mraw_v, udma_v, mdma_v, u0_v, u1_v, m0_v, m1_v,
              w_v, wb_v, o_v, sem_u, sem_m):
    wid = lax.axis_index("s") * NC + lax.axis_index("c")
    base = wid * BPW

    pltpu.sync_copy(fcw_hbm.at[0], w_v)
    for d in range(D):
        wv = w_v[pl.ds((d // 16) * 16, 16)]
        wb_v[d] = jnp.full((16,), wv[d % 16], jnp.float32)

    for c in range(NCH):
        pltpu.sync_copy(uidx_hbm.at[pl.ds(base + c * CH, CH)], uraw_v.at[c])
        pltpu.sync_copy(midx_hbm.at[pl.ds(base + c * CH, CH)], mraw_v.at[c])

    for c in range(NCH):
        for g in range(CH // 16):
            sl = pl.ds(g * 16, 16)
            udma_v[c, sl] = uraw_v[c, sl] >> 1
            mdma_v[c, sl] = mraw_v[c, sl] >> 1

    iota = lax.iota(jnp.int32, 16)
    bufs = [(u0_v, m0_v), (u1_v, m1_v)]

    def start(c, buf):
        ub, mb = bufs[buf]
        cu = pltpu.async_copy(uc_hbm.at[udma_v.at[c]], ub, sem_u)
        cm = pltpu.async_copy(mc_hbm.at[mdma_v.at[c]], mb, sem_m)
        return cu, cm

    pend = start(0, 0)
    for c in range(NCH):
        if c + 1 < NCH:
            nxt = start(c + 1, (c + 1) % 2)
        pend[0].wait()
        pend[1].wait()
        ub, mb = bufs[c % 2]

        def group(g, carry):
            i0 = iota + g * 16
            uoff = (uraw_v[c, pl.ds(g * 16, 16)] & 1) << 6
            moff = (mraw_v[c, pl.ds(g * 16, 16)] & 1) << 6
            acc = jnp.zeros((16,), jnp.float32)
            for d in range(D):
                uu = plsc.load_gather(ub, [i0, uoff + d])
                mm = plsc.load_gather(mb, [i0, moff + d])
                acc = acc + uu * mm * wb_v[d]
            o_v[pl.ds(c * CH + g * 16, 16)] = acc
            return carry

        lax.fori_loop(0, CH // 16, group, 0)

        if c + 1 < NCH:
            pend = nxt

    pltpu.sync_copy(o_v, out_hbm.at[pl.ds(base, BPW)])


_cf_dot = pl.kernel(
    _dot_body,
    mesh=_mesh,
    compiler_params=pltpu.CompilerParams(needs_layout_passes=False),
    out_type=jax.ShapeDtypeStruct((B,), jnp.float32),
    scratch_types=[
        pltpu.VMEM((NCH, CH), jnp.int32),
        pltpu.VMEM((NCH, CH), jnp.int32),
        pltpu.VMEM((NCH, CH), jnp.int32),
        pltpu.VMEM((NCH, CH), jnp.int32),
        pltpu.VMEM((CH, 2 * D), jnp.float32),
        pltpu.VMEM((CH, 2 * D), jnp.float32),
        pltpu.VMEM((CH, 2 * D), jnp.float32),
        pltpu.VMEM((CH, 2 * D), jnp.float32),
        pltpu.VMEM((192,), jnp.float32),
        pltpu.VMEM((D, 16), jnp.float32),
        pltpu.VMEM((BPW,), jnp.float32),
        pltpu.SemaphoreType.DMA,
        pltpu.SemaphoreType.DMA,
    ],
)

TB = 2048


def _tc_body(f_ref, w2_ref, b_ref, o_ref):
    o_ref[...] = jnp.sum(f_ref[...] * w2_ref[...], axis=1) + b_ref[0, 0]


_tc_feats = pl.pallas_call(
    _tc_body,
    grid=(B // TB,),
    in_specs=[
        pl.BlockSpec((TB, F), lambda i: (i, 0)),
        pl.BlockSpec((1, F), lambda i: (0, 0)),
        pl.BlockSpec((1, 1), lambda i: (0, 0)),
    ],
    out_specs=pl.BlockSpec((TB,), lambda i: (i,)),
    out_shape=jax.ShapeDtypeStruct((B,), jnp.float32),
)


def kernel(user_idx, movie_idx, movie_feats, user_table, movie_table, fc_w, fc_b):
    uc = user_table[:NU].reshape(NU // 2, 2 * D)
    mc = movie_table[:NM].reshape(NM // 2, 2 * D)
    cf = _cf_dot(user_idx, movie_idx, fc_w, uc, mc)
    w2 = fc_w[:, D:]
    b = fc_b.reshape(1, 1)
    content = _tc_feats(movie_feats, w2, b)
    return cf + content


# speedup vs baseline: 2.8898x; 1.5827x over previous
import jax
import jax.numpy as jnp
from jax import lax
from jax.experimental import pallas as pl
from jax.experimental.pallas import tpu as pltpu
from jax.experimental.pallas import tpu_sc as plsc

B = 16384
D = 64
F = 128
NC = 2
NS = 16
NW = NC * NS
BPW = B // NW
CH = 128
NCH = BPW // CH

_mesh = plsc.VectorSubcoreMesh(core_axis_name="c", subcore_axis_name="s")


def _dot_body(uidx_hbm, midx_hbm, fcw_hbm, utab_hbm, mtab_hbm, out_hbm,
              uraw_v, mraw_v, u0_v, u1_v, m0_v, m1_v,
              w_v, wb_v, o_v, sem_u, sem_m):
    wid = lax.axis_index("s") * NC + lax.axis_index("c")
    base = wid * BPW

    pltpu.sync_copy(fcw_hbm.at[0], w_v)
    for d in range(D):
        wv = w_v[pl.ds((d // 16) * 16, 16)]
        wb_v[d] = jnp.full((16,), wv[d % 16], jnp.float32)

    for c in range(NCH):
        pltpu.sync_copy(uidx_hbm.at[pl.ds(base + c * CH, CH)], uraw_v.at[c])
        pltpu.sync_copy(midx_hbm.at[pl.ds(base + c * CH, CH)], mraw_v.at[c])

    iota = lax.iota(jnp.int32, 16)
    bufs = [(u0_v, m0_v), (u1_v, m1_v)]

    def start(c, buf):
        ub, mb = bufs[buf]

        def g_body(g, carry):
            vu = uraw_v[c, pl.ds(g * 16, 16)]
            vm = mraw_v[c, pl.ds(g * 16, 16)]
            r0 = g * 16
            for l in range(16):
                pltpu.async_copy(utab_hbm.at[vu[l]], ub.at[r0 + l], sem_u)
                pltpu.async_copy(mtab_hbm.at[vm[l]], mb.at[r0 + l], sem_m)
            return carry

        lax.fori_loop(0, CH // 16, g_body, 0)

    def drain(buf):
        ub, mb = bufs[buf]
        pltpu.make_async_copy(utab_hbm.at[pl.ds(0, CH)], ub, sem_u).wait()
        pltpu.make_async_copy(mtab_hbm.at[pl.ds(0, CH)], mb, sem_m).wait()

    start(0, 0)
    for c in range(NCH):
        if c + 1 < NCH:
            start(c + 1, (c + 1) % 2)
        drain(c % 2)
        ub, mb = bufs[c % 2]

        def group(g, carry):
            i0 = iota + g * 16
            acc = jnp.zeros((16,), jnp.float32)
            for d in range(D):
                i1 = jnp.full((16,), d, jnp.int32)
                uu = plsc.load_gather(ub, [i0, i1])
                mm = plsc.load_gather(mb, [i0, i1])
                acc = acc + uu * mm * wb_v[d]
            o_v[pl.ds(c * CH + g * 16, 16)] = acc
            return carry

        lax.fori_loop(0, CH // 16, group, 0)

    pltpu.sync_copy(o_v, out_hbm.at[pl.ds(base, BPW)])


_cf_dot = pl.kernel(
    _dot_body,
    mesh=_mesh,
    compiler_params=pltpu.CompilerParams(needs_layout_passes=False),
    out_type=jax.ShapeDtypeStruct((B,), jnp.float32),
    scratch_types=[
        pltpu.VMEM((NCH, CH), jnp.int32),
        pltpu.VMEM((NCH, CH), jnp.int32),
        pltpu.VMEM((CH, D), jnp.float32),
        pltpu.VMEM((CH, D), jnp.float32),
        pltpu.VMEM((CH, D), jnp.float32),
        pltpu.VMEM((CH, D), jnp.float32),
        pltpu.VMEM((192,), jnp.float32),
        pltpu.VMEM((D, 16), jnp.float32),
        pltpu.VMEM((BPW,), jnp.float32),
        pltpu.SemaphoreType.DMA,
        pltpu.SemaphoreType.DMA,
    ],
)

TB = 2048


def _tc_body(f_ref, w2_ref, b_ref, o_ref):
    o_ref[...] = jnp.sum(f_ref[...] * w2_ref[...], axis=1) + b_ref[0, 0]


_tc_feats = pl.pallas_call(
    _tc_body,
    grid=(B // TB,),
    in_specs=[
        pl.BlockSpec((TB, F), lambda i: (i, 0)),
        pl.BlockSpec((1, F), lambda i: (0, 0)),
        pl.BlockSpec((1, 1), lambda i: (0, 0)),
    ],
    out_specs=pl.BlockSpec((TB,), lambda i: (i,)),
    out_shape=jax.ShapeDtypeStruct((B,), jnp.float32),
)


def kernel(user_idx, movie_idx, movie_feats, user_table, movie_table, fc_w, fc_b):
    cf = _cf_dot(user_idx, movie_idx, fc_w, user_table, movie_table)
    w2 = fc_w[:, D:]
    b = fc_b.reshape(1, 1)
    content = _tc_feats(movie_feats, w2, b)
    return cf + content


# speedup vs baseline: 2.9612x; 1.0247x over previous
import jax
import jax.numpy as jnp
from jax import lax
from jax.experimental import pallas as pl
from jax.experimental.pallas import tpu as pltpu
from jax.experimental.pallas import tpu_sc as plsc

B = 16384
D = 64
F = 128
NC = 2
NS = 16
NW = NC * NS
BPW = B // NW
CH = 128
NCH = BPW // CH

_mesh = plsc.VectorSubcoreMesh(core_axis_name="c", subcore_axis_name="s")


def _sc_body(uidx_hbm, midx_hbm, utab_hbm, mtab_hbm, p_hbm,
             idx_v, u_v, m_v, sem_u, sem_m):
    wid = lax.axis_index("s") * NC + lax.axis_index("c")
    base = wid * BPW
    pltpu.sync_copy(uidx_hbm.at[pl.ds(base, BPW)], idx_v.at[0])
    pltpu.sync_copy(midx_hbm.at[pl.ds(base, BPW)], idx_v.at[1])
    for c in range(NCH):
        copies = []
        for g in range(CH // 16):
            r0 = c * CH + g * 16
            vu = idx_v[0, pl.ds(r0, 16)]
            vm = idx_v[1, pl.ds(r0, 16)]
            for l in range(16):
                copies.append(pltpu.async_copy(
                    utab_hbm.at[vu[l]], u_v.at[g * 16 + l], sem_u))
                copies.append(pltpu.async_copy(
                    mtab_hbm.at[vm[l]], m_v.at[g * 16 + l], sem_m))
        for cp in copies:
            cp.wait()

        def mul_row(r, carry):
            for j in range(D // 16):
                sl = pl.ds(j * 16, 16)
                u_v[r, sl] = u_v[r, sl] * m_v[r, sl]
            return carry

        lax.fori_loop(0, CH, mul_row, 0)
        pltpu.sync_copy(u_v, p_hbm.at[pl.ds(base + c * CH, CH)])


_sc_gather_mul = pl.kernel(
    _sc_body,
    mesh=_mesh,
    out_type=jax.ShapeDtypeStruct((B, D), jnp.float32),
    scratch_types=[
        pltpu.VMEM((2, BPW), jnp.int32),
        pltpu.VMEM((CH, D), jnp.float32),
        pltpu.VMEM((CH, D), jnp.float32),
        pltpu.SemaphoreType.DMA,
        pltpu.SemaphoreType.DMA,
    ],
)

TB = 2048


def _tc_body(um_ref, f_ref, w1_ref, w2_ref, b_ref, o_ref):
    o_ref[...] = (
        jnp.dot(um_ref[...], w1_ref[...], preferred_element_type=jnp.float32)
        + jnp.dot(f_ref[...], w2_ref[...], preferred_element_type=jnp.float32)
        + b_ref[...]
    )


_tc_call = pl.pallas_call(
    _tc_body,
    grid=(B // TB,),
    in_specs=[
        pl.BlockSpec((TB, D), lambda i: (i, 0)),
        pl.BlockSpec((TB, F), lambda i: (i, 0)),
        pl.BlockSpec((D, 1), lambda i: (0, 0)),
        pl.BlockSpec((F, 1), lambda i: (0, 0)),
        pl.BlockSpec((1, 1), lambda i: (0, 0)),
    ],
    out_specs=pl.BlockSpec((TB, 1), lambda i: (i, 0)),
    out_shape=jax.ShapeDtypeStruct((B, 1), jnp.float32),
)


def kernel(user_idx, movie_idx, movie_feats, user_table, movie_table, fc_w, fc_b):
    p = _sc_gather_mul(user_idx, movie_idx, user_table, movie_table)
    w1 = fc_w[:, :D].T
    w2 = fc_w[:, D:].T
    b = fc_b.reshape(1, 1)
    out2 = _tc_call(p, movie_feats, w1, w2, b)
    return out2[:, 0]
